# baseline (speedup 1.0000x reference)
import functools

import jax
import jax.numpy as jnp
from jax import lax
from jax.experimental import pallas as pl
from jax.experimental.pallas import tpu as pltpu
from jax.experimental.pallas import tpu_sc as plsc

H = 2048
M = 1408
MS = 5632
E = 16
T = 32

P1C = 128
NP1 = H // P1C
EHC = 512
NEH = H // EHC
DCH = H // NEH
SKC = 256
NSK = MS // SKC

NE = E * NEH
NSTEPS = NE + NEH
J_ACT = 17
J_P2 = NSTEPS - NSK


def _sc_router(flat_hbm, rw_hbm, logits_hbm, row_v, rw_v, out_v):
    wid = lax.axis_index("s") * 2 + lax.axis_index("c")
    pltpu.sync_copy(flat_hbm.at[wid], row_v)
    acc = jnp.zeros((E,), jnp.float32)
    for c in range(H // 128):
        pltpu.sync_copy(rw_hbm.at[pl.ds(c * 128, 128)], rw_v)

        def step(i, acc, _c=c):
            vals = row_v[pl.ds(_c * 128 + i * 16, 16)]
            for l in range(16):
                acc = acc + vals[l] * rw_v[i * 16 + l]
            return acc

        acc = lax.fori_loop(0, 8, step, acc)
    out_v[...] = acc
    pltpu.sync_copy(out_v, logits_hbm.at[wid])


def _body(flat_ref, rw_ref, eg_ref, sg_ref, si_ref, gate_ref, outw_ref,
          so_ref, out_ref, logits_ref, g_ref, x_ref, h_sh_ref, seg_ref,
          rout_ref, gu_ref, he_ref):
    j = pl.program_id(0)

    @pl.when(j == 0)
    def _router():
        flat = flat_ref[...]
        logits = jnp.dot(flat, rw_ref[...], preferred_element_type=jnp.float32)
        logits_ref[...] = logits
        m = jnp.max(logits, axis=-1, keepdims=True)
        ex = jnp.exp(logits - m)
        probs = ex / jnp.sum(ex, axis=-1, keepdims=True)
        lane = jax.lax.broadcasted_iota(jnp.int32, probs.shape, 1)
        p1 = jnp.max(probs, axis=-1, keepdims=True)
        i1 = jnp.min(jnp.where(probs == p1, lane, E), axis=-1, keepdims=True)
        is1 = lane == i1
        probs2 = jnp.where(is1, -1.0, probs)
        p2 = jnp.max(probs2, axis=-1, keepdims=True)
        i2 = jnp.min(jnp.where(probs2 == p2, lane, E), axis=-1, keepdims=True)
        is2 = lane == i2
        s = p1 + p2
        rout_ref[...] = (jnp.where(is1, p1 / s, 0.0)
                         + jnp.where(is2, p2 / s, 0.0))
        seg_ref[...] = jnp.dot(flat, eg_ref[...],
                               preferred_element_type=jnp.float32)

    @pl.when(j < NP1)
    def _phase1():
        fc = flat_ref[:, pl.ds(j * P1C, P1C)]
        gp = jnp.dot(fc, sg_ref[...], preferred_element_type=jnp.float32)
        xp = jnp.dot(fc, si_ref[...], preferred_element_type=jnp.float32)

        @pl.when(j == 0)
        def _reset():
            g_ref[...] = gp
            x_ref[...] = xp

        @pl.when(j != 0)
        def _accum():
            g_ref[...] += gp
            x_ref[...] += xp

    @pl.when(j == J_ACT)
    def _activate():
        g = g_ref[...]
        h_sh_ref[...] = x_ref[...] * (g * jax.nn.sigmoid(g)) * \
            jax.nn.sigmoid(seg_ref[...])

    @pl.when(j >= NEH)
    def _down():
        kd = j - NEH
        cd = kd % NEH
        contrib = jnp.dot(he_ref[...], outw_ref[0],
                          preferred_element_type=jnp.float32)

        @pl.when(kd < NEH)
        def _init():
            out_ref[:, pl.ds(cd * DCH, DCH)] = contrib

        @pl.when(kd >= NEH)
        def _add():
            out_ref[:, pl.ds(cd * DCH, DCH)] += contrib

    @pl.when(j < NE)
    def _expert():
        k = j % NEH
        fc = flat_ref[:, pl.ds(k * EHC, EHC)]
        part = jnp.dot(fc, gate_ref[0], preferred_element_type=jnp.float32)

        @pl.when(k == 0)
        def _reset():
            gu_ref[...] = part

        @pl.when(k != 0)
        def _accum():
            gu_ref[...] += part

        @pl.when(k == NEH - 1)
        def _act_e():
            e = j // NEH
            gu = gu_ref[...]
            g = gu[:, :M]
            u = gu[:, M:]
            lane = jax.lax.broadcasted_iota(jnp.int32, (T, E), 1)
            w = jnp.sum(jnp.where(lane == e, rout_ref[...], 0.0), axis=1,
                        keepdims=True)
            he_ref[...] = (g * jax.nn.sigmoid(g)) * u * w

    @pl.when(j >= J_P2)
    def _phase2():
        ks = j - J_P2
        hc = h_sh_ref[:, pl.ds(ks * SKC, SKC)]
        out_ref[...] += jnp.dot(hc, so_ref[...],
                                preferred_element_type=jnp.float32)


def kernel(hidden_states, router_w, expert_gate_w, expert_out_w,
           shared_gate_w, shared_inter_w, shared_out_w, shared_eg_w):
    B, S, _ = hidden_states.shape
    flat = hidden_states.reshape(-1, H)

    logits_sc = functools.partial(
        pl.kernel,
        out_type=jax.ShapeDtypeStruct((T, E), jnp.float32),
        mesh=plsc.VectorSubcoreMesh(core_axis_name="c", subcore_axis_name="s"),
        scratch_types=[
            pltpu.VMEM((H,), jnp.float32),
            pltpu.VMEM((128, E), jnp.float32),
            pltpu.VMEM((E,), jnp.float32),
        ],
    )(_sc_router)(flat, router_w)

    def _e_idx(j):
        ke = jnp.clip(j, 0, NE - 1)
        return (ke // NEH, ke % NEH, 0)

    def _d_idx(j):
        kd = jnp.clip(j - NEH, 0, NE - 1)
        return (kd // NEH, 0, kd % NEH)

    out_flat, logits = pl.pallas_call(
        _body,
        grid=(NSTEPS,),
        in_specs=[
            pl.BlockSpec((T, H), lambda j: (0, 0)),
            pl.BlockSpec((H, E), lambda j: (0, 0)),
            pl.BlockSpec((H, 1), lambda j: (0, 0)),
            pl.BlockSpec((P1C, MS), lambda j: (jnp.clip(j, 0, NP1 - 1), 0)),
            pl.BlockSpec((P1C, MS), lambda j: (jnp.clip(j, 0, NP1 - 1), 0)),
            pl.BlockSpec((1, EHC, 2 * M), _e_idx),
            pl.BlockSpec((1, M, DCH), _d_idx),
            pl.BlockSpec((SKC, H), lambda j: (jnp.clip(j - J_P2, 0, NSK - 1), 0)),
        ],
        out_specs=(
            pl.BlockSpec((T, H), lambda j: (0, 0)),
            pl.BlockSpec((T, E), lambda j: (0, 0)),
        ),
        out_shape=(
            jax.ShapeDtypeStruct((T, H), jnp.float32),
            jax.ShapeDtypeStruct((T, E), jnp.float32),
        ),
        scratch_shapes=[
            pltpu.VMEM((T, MS), jnp.float32),
            pltpu.VMEM((T, MS), jnp.float32),
            pltpu.VMEM((T, MS), jnp.float32),
            pltpu.VMEM((T, 1), jnp.float32),
            pltpu.VMEM((T, E), jnp.float32),
            pltpu.VMEM((T, 2 * M), jnp.float32),
            pltpu.VMEM((T, M), jnp.float32),
        ],
        compiler_params=pltpu.CompilerParams(
            dimension_semantics=("arbitrary",)),
    )(flat, router_w, shared_eg_w, shared_gate_w, shared_inter_w,
      expert_gate_w, expert_out_w, shared_out_w)

    return (out_flat.reshape(B, S, H), logits_sc)


# speedup vs baseline: 1.1771x; 1.1771x over previous
import jax
import jax.numpy as jnp
from jax.experimental import pallas as pl
from jax.experimental.pallas import tpu as pltpu

H = 2048
M = 1408
MS = 5632
E = 16
T = 32

P1C = 128
NP1 = H // P1C
EHC = 512
NEH = H // EHC
DCH = H // NEH
SKC = 256
NSK = MS // SKC

NE = E * NEH
NSTEPS = NE + NEH
J_ACT = 17
J_P2 = NSTEPS - NSK


def _body(flat_ref, rw_ref, eg_ref, sg_ref, si_ref, gate_ref, outw_ref,
          so_ref, out_ref, logits_ref, g_ref, x_ref, h_sh_ref, seg_ref,
          rout_ref, gu_ref, he_ref):
    j = pl.program_id(0)

    @pl.when(j == 0)
    def _router():
        flat = flat_ref[...]
        logits = jnp.dot(flat, rw_ref[...], preferred_element_type=jnp.float32)
        logits_ref[...] = logits
        m = jnp.max(logits, axis=-1, keepdims=True)
        ex = jnp.exp(logits - m)
        probs = ex / jnp.sum(ex, axis=-1, keepdims=True)
        lane = jax.lax.broadcasted_iota(jnp.int32, probs.shape, 1)
        p1 = jnp.max(probs, axis=-1, keepdims=True)
        i1 = jnp.min(jnp.where(probs == p1, lane, E), axis=-1, keepdims=True)
        is1 = lane == i1
        probs2 = jnp.where(is1, -1.0, probs)
        p2 = jnp.max(probs2, axis=-1, keepdims=True)
        i2 = jnp.min(jnp.where(probs2 == p2, lane, E), axis=-1, keepdims=True)
        is2 = lane == i2
        s = p1 + p2
        rout_ref[...] = (jnp.where(is1, p1 / s, 0.0)
                         + jnp.where(is2, p2 / s, 0.0))
        seg_ref[...] = jnp.dot(flat, eg_ref[...],
                               preferred_element_type=jnp.float32)

    @pl.when(j < NP1)
    def _phase1():
        fc = flat_ref[:, pl.ds(j * P1C, P1C)]
        gp = jnp.dot(fc, sg_ref[...], preferred_element_type=jnp.float32)
        xp = jnp.dot(fc, si_ref[...], preferred_element_type=jnp.float32)

        @pl.when(j == 0)
        def _reset():
            g_ref[...] = gp
            x_ref[...] = xp

        @pl.when(j != 0)
        def _accum():
            g_ref[...] += gp
            x_ref[...] += xp

    @pl.when(j == J_ACT)
    def _activate():
        g = g_ref[...]
        h_sh_ref[...] = x_ref[...] * (g * jax.nn.sigmoid(g)) * \
            jax.nn.sigmoid(seg_ref[...])

    @pl.when(j >= NEH)
    def _down():
        kd = j - NEH
        cd = kd % NEH
        contrib = jnp.dot(he_ref[...], outw_ref[0],
                          preferred_element_type=jnp.float32)

        @pl.when(kd < NEH)
        def _init():
            out_ref[:, pl.ds(cd * DCH, DCH)] = contrib

        @pl.when(kd >= NEH)
        def _add():
            out_ref[:, pl.ds(cd * DCH, DCH)] += contrib

    @pl.when(j < NE)
    def _expert():
        k = j % NEH
        fc = flat_ref[:, pl.ds(k * EHC, EHC)]
        part = jnp.dot(fc, gate_ref[0], preferred_element_type=jnp.float32)

        @pl.when(k == 0)
        def _reset():
            gu_ref[...] = part

        @pl.when(k != 0)
        def _accum():
            gu_ref[...] += part

        @pl.when(k == NEH - 1)
        def _act_e():
            e = j // NEH
            gu = gu_ref[...]
            g = gu[:, :M]
            u = gu[:, M:]
            lane = jax.lax.broadcasted_iota(jnp.int32, (T, E), 1)
            w = jnp.sum(jnp.where(lane == e, rout_ref[...], 0.0), axis=1,
                        keepdims=True)
            he_ref[...] = (g * jax.nn.sigmoid(g)) * u * w

    @pl.when(j >= J_P2)
    def _phase2():
        ks = j - J_P2
        hc = h_sh_ref[:, pl.ds(ks * SKC, SKC)]
        out_ref[...] += jnp.dot(hc, so_ref[...],
                                preferred_element_type=jnp.float32)


def kernel(hidden_states, router_w, expert_gate_w, expert_out_w,
           shared_gate_w, shared_inter_w, shared_out_w, shared_eg_w):
    B, S, _ = hidden_states.shape
    flat = hidden_states.reshape(-1, H)

    def _e_idx(j):
        ke = jnp.clip(j, 0, NE - 1)
        return (ke // NEH, ke % NEH, 0)

    def _d_idx(j):
        kd = jnp.clip(j - NEH, 0, NE - 1)
        return (kd // NEH, 0, kd % NEH)

    out_flat, logits = pl.pallas_call(
        _body,
        grid=(NSTEPS,),
        in_specs=[
            pl.BlockSpec((T, H), lambda j: (0, 0)),
            pl.BlockSpec((H, E), lambda j: (0, 0)),
            pl.BlockSpec((H, 1), lambda j: (0, 0)),
            pl.BlockSpec((P1C, MS), lambda j: (jnp.clip(j, 0, NP1 - 1), 0)),
            pl.BlockSpec((P1C, MS), lambda j: (jnp.clip(j, 0, NP1 - 1), 0)),
            pl.BlockSpec((1, EHC, 2 * M), _e_idx),
            pl.BlockSpec((1, M, DCH), _d_idx),
            pl.BlockSpec((SKC, H), lambda j: (jnp.clip(j - J_P2, 0, NSK - 1), 0)),
        ],
        out_specs=(
            pl.BlockSpec((T, H), lambda j: (0, 0)),
            pl.BlockSpec((T, E), lambda j: (0, 0)),
        ),
        out_shape=(
            jax.ShapeDtypeStruct((T, H), jnp.float32),
            jax.ShapeDtypeStruct((T, E), jnp.float32),
        ),
        scratch_shapes=[
            pltpu.VMEM((T, MS), jnp.float32),
            pltpu.VMEM((T, MS), jnp.float32),
            pltpu.VMEM((T, MS), jnp.float32),
            pltpu.VMEM((T, 1), jnp.float32),
            pltpu.VMEM((T, E), jnp.float32),
            pltpu.VMEM((T, 2 * M), jnp.float32),
            pltpu.VMEM((T, M), jnp.float32),
        ],
        compiler_params=pltpu.CompilerParams(
            dimension_semantics=("arbitrary",)),
    )(flat, router_w, shared_eg_w, shared_gate_w, shared_inter_w,
      expert_gate_w, expert_out_w, shared_out_w)

    return (out_flat.reshape(B, S, H), logits)
